# baseline (speedup 1.0000x reference)
import functools

import jax
import jax.numpy as jnp
from jax import lax
from jax.experimental import pallas as pl
from jax.experimental.pallas import tpu as pltpu
from jax.experimental.pallas import tpu_sc as plsc

DIM = 50
DPAD = 64
BATCH = 16384
GAMMA = 1.0

NC = 2
NS = 16
L = 16
NW = NC * NS
BPW = BATCH // NW
NGRP = BPW // L
CHUNK = 128
NCHUNK = BPW // CHUNK
NVPR = DPAD // L


def _sc_kernel(ent_hbm, rel_hbm, idx_hbm, out_hbm,
               idx_v, h_buf, r_buf, t_buf, acc_all, sem):
  wid = lax.axis_index("s") * NC + lax.axis_index("c")
  pltpu.sync_copy(idx_hbm.at[wid], idx_v)

  def gather_phase(phase):
    copies = []
    for j in range(NCHUNK):
      sl = pl.ds(j * CHUNK, CHUNK)
      row0 = phase * 3 * NCHUNK
      copies.append(pltpu.async_copy(
          ent_hbm.at[idx_v.at[row0 + j]], h_buf.at[sl], sem))
      copies.append(pltpu.async_copy(
          rel_hbm.at[idx_v.at[row0 + NCHUNK + j]], r_buf.at[sl], sem))
      copies.append(pltpu.async_copy(
          ent_hbm.at[idx_v.at[row0 + 2 * NCHUNK + j]], t_buf.at[sl], sem))
    for c in copies:
      c.wait()

  def compute_phase(phase):
    def comp_grp(g, carry):
      for j in range(L):
        row = g * L + j
        acc = jnp.zeros((L,), jnp.float32)
        for k in range(NVPR):
          sl = pl.ds(k * L, L)
          acc = acc + jnp.abs(h_buf[row, sl] + r_buf[row, sl]
                              - t_buf[row, sl])
        acc_all[pl.ds((phase * BPW + row) * L, L)] = acc
      return carry
    lax.fori_loop(0, NGRP, comp_grp, jnp.int32(0))

  for phase in range(2):
    gather_phase(phase)
    compute_phase(phase)

  def diff_q(q, carry):
    d = acc_all[pl.ds(q * L, L)] - acc_all[pl.ds(L * BPW + q * L, L)]
    acc_all[pl.ds(q * L, L)] = d
    return carry
  lax.fori_loop(0, BPW, diff_q, jnp.int32(0))
  pltpu.sync_copy(acc_all.at[pl.ds(0, L * BPW)], out_hbm.at[wid])


def _tc_finish_kernel(p_ref, o_ref):
  x = p_ref[...].reshape(NW * BPW * L // 128, 128)
  r0 = lax.broadcasted_iota(jnp.int32, (128, 128 // L), 0) // L
  r1 = lax.broadcasted_iota(jnp.int32, (128, 128 // L), 1)
  m = (r0 == r1).astype(jnp.float32)
  y = lax.dot_general(x, m, (((1,), (0,)), ((), ())),
                      preferred_element_type=jnp.float32)
  o_ref[...] = jnp.sum(
      jnp.maximum(y + jnp.float32(GAMMA), jnp.float32(0.0))
  ).reshape(1, 1)


@jax.jit
def kernel(pos_head, pos_relation, pos_tail, neg_head, neg_relation, neg_tail,
           entity_emb, relation_emb):
  entp = jnp.pad(entity_emb, ((0, 0), (0, DPAD - DIM)))
  relp = jnp.pad(relation_emb, ((0, 0), (0, DPAD - DIM)))

  packed = jnp.stack([pos_head, pos_relation, pos_tail,
                      neg_head, neg_relation, neg_tail]).astype(jnp.int32)
  packed = (packed.reshape(6, NW, NCHUNK, CHUNK)
            .transpose(1, 0, 2, 3)
            .reshape(NW, 6 * NCHUNK, CHUNK))

  mesh = plsc.VectorSubcoreMesh(core_axis_name="c", subcore_axis_name="s")
  sc = pl.kernel(
      _sc_kernel,
      out_type=jax.ShapeDtypeStruct((NW, L * BPW), jnp.float32),
      mesh=mesh,
      compiler_params=pltpu.CompilerParams(
          needs_layout_passes=False,
          use_tc_tiling_on_sc=False,
      ),
      scratch_types=[
          pltpu.VMEM((6 * NCHUNK, CHUNK), jnp.int32),
          pltpu.VMEM((BPW, DPAD), jnp.float32),
          pltpu.VMEM((BPW, DPAD), jnp.float32),
          pltpu.VMEM((BPW, DPAD), jnp.float32),
          pltpu.VMEM((2 * L * BPW,), jnp.float32),
          pltpu.SemaphoreType.DMA,
      ],
  )
  partials = sc(entp, relp, packed)

  total = pl.pallas_call(
      _tc_finish_kernel,
      out_shape=jax.ShapeDtypeStruct((1, 1), jnp.float32),
  )(partials)
  return total[0, 0]
